# baseline (speedup 1.0000x reference)
import jax
import jax.numpy as jnp
from jax.experimental import pallas as pl
from jax.experimental.pallas import tpu as pltpu

_EMBED = 128
_Q = 65536
_B = 4096
_C = 16384
_R = 128
_NRB = _EMBED // _R
_NCB = _Q // _C


def _update_body(ptr_ref, norm_ref, q_ref, out_ref):
    j = pl.program_id(1)
    ptr = ptr_ref[0]

    out_ref[...] = q_ref[...]

    @pl.when(j == ptr // _C)
    def _():
        local = pl.multiple_of(ptr % _C, _B)
        out_ref[:, pl.ds(local, _B)] = norm_ref[...].T


def kernel(norm_vec, anorm_vec, temp, anorm_feats_queue, queue_ptr):
    grid_spec = pltpu.PrefetchScalarGridSpec(
        num_scalar_prefetch=1,
        grid=(_NRB, _NCB),
        in_specs=[
            pl.BlockSpec((_B, _R), lambda i, j, ptr: (0, i)),
            pl.BlockSpec((_R, _C), lambda i, j, ptr: (i, j)),
        ],
        out_specs=pl.BlockSpec((_R, _C), lambda i, j, ptr: (i, j)),
    )
    new_queue = pl.pallas_call(
        _update_body,
        grid_spec=grid_spec,
        out_shape=jax.ShapeDtypeStruct((_EMBED, _Q), jnp.float32),
    )(queue_ptr, norm_vec, anorm_feats_queue)
    new_ptr = ((queue_ptr + _B) % _Q).astype(jnp.int32)
    loss = jnp.asarray(0.0, dtype=jnp.float32)
    return loss, new_queue, new_ptr
